# baseline (speedup 1.0000x reference)
import functools

import jax
import jax.numpy as jnp
from jax import lax
from jax.experimental import pallas as pl
from jax.experimental.pallas import tpu as pltpu
from jax.experimental.pallas import tpu_sc as plsc

VOCAB = 100000
EMBED = 128
BATCH = 4096
HIST = 50

_info = plsc.get_sparse_core_info()
NC, NS = _info.num_cores, _info.num_subcores
NW = NC * NS
HP = 56

B_PER_W = BATCH // NW
BCH = 2
CH = BCH * HP
NCH = B_PER_W // BCH
NBUF = 4
NGROUPS = NCH // NBUF


def _gather_kernel(table_hbm, idx_hbm, out_hbm, idx_v, rows_v, gsem, osem):
    wid = lax.axis_index("s") * NC + lax.axis_index("c")

    pltpu.sync_copy(idx_hbm.at[wid], idx_v)

    def start_gather(j, b):
        pltpu.async_copy(table_hbm.at[idx_v.at[j]], rows_v.at[b], gsem.at[b])

    def wait_gather(j, b):
        pltpu.make_async_copy(
            table_hbm.at[idx_v.at[j]], rows_v.at[b], gsem.at[b]
        ).wait()

    def start_out(j, b):
        bstart = wid * B_PER_W + j * BCH
        for i in range(BCH):
            pltpu.async_copy(
                rows_v.at[b, pl.ds(i * HP, HP)],
                out_hbm.at[bstart + i],
                osem.at[b],
            )

    def wait_out(j, b):
        bstart = wid * B_PER_W + j * BCH
        for i in range(BCH):
            pltpu.make_async_copy(
                rows_v.at[b, pl.ds(i * HP, HP)],
                out_hbm.at[bstart + i],
                osem.at[b],
            ).wait()

    for b in range(NBUF):
        start_gather(b, b)

    def group_body(g, issue_next):
        for b in range(NBUF):
            j = g * NBUF + b
            wait_gather(j, b)
            start_out(j, b)
            if issue_next:
                wait_out(j, b)
                start_gather(j + NBUF, b)

    lax.fori_loop(
        0,
        NGROUPS - 1,
        lambda g, c: (group_body(g, True), c)[1],
        0,
        unroll=False,
    )
    group_body(NGROUPS - 1, False)

    for b in range(NBUF):
        wait_out((NGROUPS - 1) * NBUF + b, b)


@jax.jit
def _embedding_gather(table, idx3):
    mesh = plsc.VectorSubcoreMesh(core_axis_name="c", subcore_axis_name="s")
    run = functools.partial(
        pl.kernel,
        mesh=mesh,
        out_type=jax.ShapeDtypeStruct((BATCH, HP, EMBED), jnp.float32),
        scratch_types=[
            pltpu.VMEM((NCH, CH), jnp.int32),
            pltpu.VMEM((NBUF, CH, EMBED), jnp.float32),
            pltpu.SemaphoreType.DMA((NBUF,)),
            pltpu.SemaphoreType.DMA((NBUF,)),
        ],
    )(_gather_kernel)
    return run(table, idx3)


def kernel(inputs, table):
    idx = jnp.pad(inputs.astype(jnp.int32), ((0, 0), (0, HP - HIST)))
    idx3 = idx.reshape(NW, NCH, CH)
    return _embedding_gather(table, idx3)[:, :HIST, :]


# speedup vs baseline: 1.0003x; 1.0003x over previous
import functools

import jax
import jax.numpy as jnp
from jax import lax
from jax.experimental import pallas as pl
from jax.experimental.pallas import tpu as pltpu
from jax.experimental.pallas import tpu_sc as plsc

VOCAB = 100000
EMBED = 128
BATCH = 4096
HIST = 50

_info = plsc.get_sparse_core_info()
NC, NS = _info.num_cores, _info.num_subcores
NW = NC * NS
HP = 56

B_PER_W = BATCH // NW
BCH = 2
CH = BCH * HP
NCH = B_PER_W // BCH
NBUF = 4
NGROUPS = NCH // NBUF
OUT_ROWS = BATCH * HP


def _gather_kernel(table_hbm, idx_hbm, out_hbm, idx_v, rows_v, gsem, osem):
    wid = lax.axis_index("s") * NC + lax.axis_index("c")

    pltpu.sync_copy(idx_hbm.at[wid], idx_v)

    def start_gather(j, b):
        pltpu.async_copy(table_hbm.at[idx_v.at[j]], rows_v.at[b], gsem.at[b])

    def wait_gather(j, b):
        pltpu.make_async_copy(
            table_hbm.at[idx_v.at[j]], rows_v.at[b], gsem.at[b]
        ).wait()

    def start_out(j, b):
        rstart = (wid * B_PER_W + j * BCH) * HP
        pltpu.async_copy(
            rows_v.at[b], out_hbm.at[pl.ds(rstart, CH)], osem.at[b]
        )

    def wait_out(j, b):
        rstart = (wid * B_PER_W + j * BCH) * HP
        pltpu.make_async_copy(
            rows_v.at[b], out_hbm.at[pl.ds(rstart, CH)], osem.at[b]
        ).wait()

    for b in range(NBUF):
        start_gather(b, b)

    def group_body(g, issue_next):
        for b in range(NBUF):
            j = g * NBUF + b
            wait_gather(j, b)
            start_out(j, b)
            if issue_next:
                wait_out(j, b)
                start_gather(j + NBUF, b)

    lax.fori_loop(
        0,
        NGROUPS - 1,
        lambda g, c: (group_body(g, True), c)[1],
        0,
        unroll=False,
    )
    group_body(NGROUPS - 1, False)

    for b in range(NBUF):
        wait_out((NGROUPS - 1) * NBUF + b, b)


@jax.jit
def _embedding_gather(table, idx3):
    mesh = plsc.VectorSubcoreMesh(core_axis_name="c", subcore_axis_name="s")
    run = functools.partial(
        pl.kernel,
        mesh=mesh,
        out_type=jax.ShapeDtypeStruct((OUT_ROWS, EMBED), jnp.float32),
        scratch_types=[
            pltpu.VMEM((NCH, CH), jnp.int32),
            pltpu.VMEM((NBUF, CH, EMBED), jnp.float32),
            pltpu.SemaphoreType.DMA((NBUF,)),
            pltpu.SemaphoreType.DMA((NBUF,)),
        ],
    )(_gather_kernel)
    return run(table, idx3)


def kernel(inputs, table):
    idx = jnp.pad(inputs.astype(jnp.int32), ((0, 0), (0, HP - HIST)))
    idx3 = idx.reshape(NW, NCH, CH)
    out = _embedding_gather(table, idx3)
    return out.reshape(BATCH, HP, EMBED)[:, :HIST, :]


# speedup vs baseline: 6.5595x; 6.5577x over previous
import functools

import jax
import jax.numpy as jnp
from jax import lax
from jax.experimental import pallas as pl
from jax.experimental.pallas import tpu as pltpu
from jax.experimental.pallas import tpu_sc as plsc

VOCAB = 100000
EMBED = 128
BATCH = 4096
HIST = 50

_info = plsc.get_sparse_core_info()
NC, NS = _info.num_cores, _info.num_subcores
NW = NC * NS
HP = 56

B_PER_W = BATCH // NW
BCH = 2
CH = BCH * HP
NCH = B_PER_W // BCH
NBUF = 4
NGROUPS = NCH // NBUF
OUT_ROWS = BATCH * HP


def _gather_kernel(table_hbm, idx_hbm, out_hbm, idx_v, rows_v, gsem, osem):
    wid = lax.axis_index("s") * NC + lax.axis_index("c")

    pltpu.sync_copy(idx_hbm.at[wid], idx_v)

    def start_gather(j, b):
        pltpu.async_copy(table_hbm.at[idx_v.at[j]], rows_v.at[b], gsem.at[b])

    def wait_gather(j, b):
        pltpu.make_async_copy(
            table_hbm.at[idx_v.at[j]], rows_v.at[b], gsem.at[b]
        ).wait()

    def start_out(j, b):
        rstart = (wid * B_PER_W + j * BCH) * HP
        pltpu.async_copy(
            rows_v.at[b], out_hbm.at[pl.ds(rstart, CH)], osem.at[b]
        )

    def wait_out(j, b):
        rstart = (wid * B_PER_W + j * BCH) * HP
        pltpu.make_async_copy(
            rows_v.at[b], out_hbm.at[pl.ds(rstart, CH)], osem.at[b]
        ).wait()

    for b in range(NBUF):
        start_gather(b, b)

    def group_body(g, issue_next):
        for b in range(NBUF):
            j = g * NBUF + b
            wait_gather(j, b)
            start_out(j, b)
            if issue_next:
                wait_out(j, b)
                start_gather(j + NBUF, b)

    lax.fori_loop(
        0,
        NGROUPS - 1,
        lambda g, c: (group_body(g, True), c)[1],
        0,
        unroll=False,
    )
    group_body(NGROUPS - 1, False)

    for b in range(NBUF):
        wait_out((NGROUPS - 1) * NBUF + b, b)


@jax.jit
def _embedding_gather(table, idx3):
    mesh = plsc.VectorSubcoreMesh(core_axis_name="c", subcore_axis_name="s")
    run = functools.partial(
        pl.kernel,
        mesh=mesh,
        out_type=jax.ShapeDtypeStruct((OUT_ROWS, EMBED), jnp.float32),
        scratch_types=[
            pltpu.VMEM((NCH, CH), jnp.int32),
            pltpu.VMEM((NBUF, CH, EMBED), jnp.float32),
            pltpu.SemaphoreType.DMA((NBUF,)),
            pltpu.SemaphoreType.DMA((NBUF,)),
        ],
    )(_gather_kernel)
    return run(table, idx3)


def kernel(inputs, table):
    idx32 = inputs.astype(jnp.int32)
    idx = jnp.concatenate([idx32, idx32[:, : HP - HIST]], axis=1)
    idx3 = idx.reshape(NW, NCH, CH)
    out = _embedding_gather(table, idx3)
    return out.reshape(BATCH, HP, EMBED)[:, :HIST, :]


# speedup vs baseline: 7.6791x; 1.1707x over previous
import functools

import jax
import jax.numpy as jnp
from jax import lax
from jax.experimental import pallas as pl
from jax.experimental.pallas import tpu as pltpu
from jax.experimental.pallas import tpu_sc as plsc

VOCAB = 100000
EMBED = 128
BATCH = 4096
HIST = 50

_info = plsc.get_sparse_core_info()
NC, NS = _info.num_cores, _info.num_subcores
NW = NC * NS
HP = 56

B_PER_W = BATCH // NW
BCH = 2
CH = BCH * HP
NCH = B_PER_W // BCH
NBUF = 4
NGROUPS = NCH // NBUF
OUT_ROWS = BATCH * HP


def _gather_kernel(table_hbm, idx_hbm, out_hbm, idx_v, rows_v, gsem, osem):
    wid = lax.axis_index("s") * NC + lax.axis_index("c")

    pltpu.sync_copy(idx_hbm.at[wid], idx_v)

    def start_gather(j, b):
        pltpu.async_copy(table_hbm.at[idx_v.at[j]], rows_v.at[b], gsem.at[b])

    def wait_gather(j, b):
        pltpu.make_async_copy(
            table_hbm.at[idx_v.at[j]], rows_v.at[b], gsem.at[b]
        ).wait()

    def start_out(j, b):
        bstart = wid * B_PER_W + j * BCH
        for i in range(BCH):
            pltpu.async_copy(
                rows_v.at[b, pl.ds(i * HP, HIST)],
                out_hbm.at[bstart + i],
                osem.at[b],
            )

    def wait_out(j, b):
        bstart = wid * B_PER_W + j * BCH
        for i in range(BCH):
            pltpu.make_async_copy(
                rows_v.at[b, pl.ds(i * HP, HIST)],
                out_hbm.at[bstart + i],
                osem.at[b],
            ).wait()

    for b in range(NBUF):
        start_gather(b, b)

    def group_body(g, issue_next):
        for b in range(NBUF):
            j = g * NBUF + b
            wait_gather(j, b)
            start_out(j, b)
            if issue_next:
                wait_out(j, b)
                start_gather(j + NBUF, b)

    lax.fori_loop(
        0,
        NGROUPS - 1,
        lambda g, c: (group_body(g, True), c)[1],
        0,
        unroll=False,
    )
    group_body(NGROUPS - 1, False)

    for b in range(NBUF):
        wait_out((NGROUPS - 1) * NBUF + b, b)


@jax.jit
def _embedding_gather(table, idx3):
    mesh = plsc.VectorSubcoreMesh(core_axis_name="c", subcore_axis_name="s")
    run = functools.partial(
        pl.kernel,
        mesh=mesh,
        out_type=jax.ShapeDtypeStruct((BATCH, HIST, EMBED), jnp.float32),
        scratch_types=[
            pltpu.VMEM((NCH, CH), jnp.int32),
            pltpu.VMEM((NBUF, CH, EMBED), jnp.float32),
            pltpu.SemaphoreType.DMA((NBUF,)),
            pltpu.SemaphoreType.DMA((NBUF,)),
        ],
    )(_gather_kernel)
    return run(table, idx3)


def kernel(inputs, table):
    idx32 = inputs.astype(jnp.int32)
    idx = jnp.concatenate([idx32, idx32[:, : HP - HIST]], axis=1)
    idx3 = idx.reshape(NW, NCH, CH)
    return _embedding_gather(table, idx3)


# speedup vs baseline: 7.9527x; 1.0356x over previous
import functools

import jax
import jax.numpy as jnp
from jax import lax
from jax.experimental import pallas as pl
from jax.experimental.pallas import tpu as pltpu
from jax.experimental.pallas import tpu_sc as plsc

VOCAB = 100000
EMBED = 128
BATCH = 4096
HIST = 50

_info = plsc.get_sparse_core_info()
NC, NS = _info.num_cores, _info.num_subcores
NW = NC * NS

B_PER_W = BATCH // NW
BCH = 2
CH = BCH * HIST
CHS = 104
NCH = B_PER_W // BCH
NBUF = 8
NGROUPS = NCH // NBUF


def _gather_kernel(table_hbm, idx_hbm, out_hbm, idx_v, rows_v, gsem, osem):
    wid = lax.axis_index("s") * NC + lax.axis_index("c")

    pltpu.sync_copy(idx_hbm.at[wid], idx_v)

    def start_gather(j, b):
        pltpu.async_copy(
            table_hbm.at[idx_v.at[j, pl.ds(0, CH)]], rows_v.at[b], gsem.at[b]
        )

    def wait_gather(j, b):
        pltpu.make_async_copy(
            table_hbm.at[idx_v.at[j, pl.ds(0, CH)]], rows_v.at[b], gsem.at[b]
        ).wait()

    def start_out(j, b):
        bstart = wid * B_PER_W + j * BCH
        for i in range(BCH):
            pltpu.async_copy(
                rows_v.at[b, pl.ds(i * HIST, HIST)],
                out_hbm.at[bstart + i],
                osem.at[b],
            )

    def wait_out(j, b):
        bstart = wid * B_PER_W + j * BCH
        for i in range(BCH):
            pltpu.make_async_copy(
                rows_v.at[b, pl.ds(i * HIST, HIST)],
                out_hbm.at[bstart + i],
                osem.at[b],
            ).wait()

    for b in range(NBUF):
        start_gather(b, b)

    def group_body(g, issue_next):
        for b in range(NBUF):
            j = g * NBUF + b
            wait_gather(j, b)
            start_out(j, b)
            if issue_next:
                wait_out(j, b)
                start_gather(j + NBUF, b)

    lax.fori_loop(
        0,
        NGROUPS - 1,
        lambda g, c: (group_body(g, True), c)[1],
        0,
        unroll=False,
    )
    group_body(NGROUPS - 1, False)

    for b in range(NBUF):
        wait_out((NGROUPS - 1) * NBUF + b, b)


@jax.jit
def _embedding_gather(table, idx3):
    mesh = plsc.VectorSubcoreMesh(core_axis_name="c", subcore_axis_name="s")
    run = functools.partial(
        pl.kernel,
        mesh=mesh,
        out_type=jax.ShapeDtypeStruct((BATCH, HIST, EMBED), jnp.float32),
        scratch_types=[
            pltpu.VMEM((NCH, CHS), jnp.int32),
            pltpu.VMEM((NBUF, CH, EMBED), jnp.float32),
            pltpu.SemaphoreType.DMA((NBUF,)),
            pltpu.SemaphoreType.DMA((NBUF,)),
        ],
    )(_gather_kernel)
    return run(table, idx3)


def kernel(inputs, table):
    idx2 = inputs.astype(jnp.int32).reshape(BATCH // BCH, CH)
    idx2 = jnp.pad(idx2, ((0, 0), (0, CHS - CH)))
    idx3 = idx2.reshape(NW, NCH, CHS)
    return _embedding_gather(table, idx3)
